# Initial kernel scaffold; baseline (speedup 1.0000x reference)
import jax
import jax.numpy as jnp
from jax.experimental import pallas as pl


def kernel(cell_feat, sub_feat, maps, lin_w0, lin_b0, lin_w1, lin_b1, prelu_w, bn_w, bn_b):
    raise NotImplementedError("write your pallas kernel here")


# speedup vs baseline: 2217.3923x; 2217.3923x over previous
import jax
import jax.numpy as jnp
from jax.experimental import pallas as pl


def _graph_conv_kernel(x_ref, maps_ref, w0_ref, b0_ref, w1_ref, b1_ref,
                       pw_ref, bnw_ref, bnb_ref, out_ref):
    maps = maps_ref[...]
    m = jnp.where(maps != 0.5, maps, 0.0)
    a = m + m.T
    deg = jnp.sum(a, axis=1, keepdims=True) + 1.0
    degc = jnp.maximum(deg, 1e-12)
    r = jax.lax.rsqrt(degc)
    r = r * (1.5 - 0.5 * degc * r * r)
    r = r * (1.5 - 0.5 * degc * r * r)
    dis = jnp.where(deg > 0.0, r, 0.0)

    def hop(h):
        z = dis * h
        az = jax.lax.dot_general(a, z, (((1,), (0,)), ((), ())),
                                 preferred_element_type=jnp.float32,
                                 precision=jax.lax.Precision.HIGHEST)
        return dis * (az + z)

    pw = pw_ref[...]
    bnw = bnw_ref[...]
    bnb = bnb_ref[...]

    x = x_ref[...]
    for w_ref, b_ref in ((w0_ref, b0_ref), (w1_ref, b1_ref)):
        h = hop(hop(x))
        hq = h.astype(jnp.bfloat16).astype(jnp.float32)
        wq = w_ref[...].astype(jnp.bfloat16).astype(jnp.float32)
        h = jax.lax.dot_general(hq, wq, (((1,), (1,)), ((), ())),
                                preferred_element_type=jnp.float32,
                                precision=jax.lax.Precision.HIGHEST)
        h = h + b_ref[...]
        h = jnp.where(h >= 0.0, h, pw * h)
        mean = jnp.mean(h, axis=0, keepdims=True)
        var = jnp.mean((h - mean) * (h - mean), axis=0, keepdims=True)
        v = var + 1e-5
        s = jax.lax.rsqrt(v)
        s = s * (1.5 - 0.5 * v * s * s)
        s = s * (1.5 - 0.5 * v * s * s)
        x = (h - mean) * s * bnw + bnb
    out_ref[...] = x


def kernel(cell_feat, sub_feat, maps, lin_w0, lin_b0, lin_w1, lin_b1,
           prelu_w, bn_w, bn_b):
    n_cell = cell_feat.shape[0]
    c = cell_feat.shape[1]
    x = jnp.concatenate([cell_feat, sub_feat], axis=0)
    n = x.shape[0]

    out = pl.pallas_call(
        _graph_conv_kernel,
        out_shape=jax.ShapeDtypeStruct((n, c), jnp.float32),
    )(x, maps,
      lin_w0, lin_b0.reshape(1, c),
      lin_w1, lin_b1.reshape(1, c),
      prelu_w.reshape(1, c), bn_w.reshape(1, c), bn_b.reshape(1, c))

    return out[:n_cell, :], out[n_cell:, :]
